# baseline (speedup 1.0000x reference)
import functools

import jax
import jax.numpy as jnp
from jax import lax
from jax.experimental import pallas as pl
from jax.experimental.pallas import tpu as pltpu
from jax.experimental.pallas import tpu_sc as plsc

N = 50000
D = 128
E = 500000
NB = 64
NCLS = 10
NC, NS = 2, 16
BR = 512
NP = 50176
GRID = NP // BR
ECH = 128
G = 3
GE = G * ECH
NPAIR = 41
EPAD = NS * GE * 2 * NPAIR
NCHUNK = EPAD // (NS * ECH)
RPS = NP // NS
FC = 4
FW = 32

_f32 = jnp.float32
_i32 = jnp.int32


def _sc_body(with_cnt, *refs):
    nouts = 2 if with_cnt else 1
    (tab, s4, dref, zs) = refs[:4]
    outs = refs[4:4 + nouts]
    (spmem, isrc, idst, rows, sem_i, sem_g, sem_s, sem_z) = refs[4 + nouts:]
    aggref = outs[0]
    cntref = outs[1] if with_cnt else None

    c = lax.axis_index("c")
    s = lax.axis_index("s")
    base = s * RPS

    ov = jnp.ones((16,), _f32)

    def fill1(r, _):
        rows[0, 0, r, pl.ds(0, 16)] = ov
        rows[0, 0, r, pl.ds(16, 16)] = ov
        return 0

    def zero_stripe():
        pltpu.async_copy(zs, spmem.at[pl.ds(base, RPS)], sem_z).wait()

    def _load_idx(gather, g, p):
        off = s * (NCHUNK * ECH) + g * GE
        if gather:
            pltpu.async_copy(s4.at[pl.ds(off, GE)], isrc.at[p], sem_i)
        pltpu.async_copy(dref.at[pl.ds(s * NCHUNK + g * G, G)], idst.at[p], sem_i)

    def _wait_idx(gather, fofs, g, p):
        off = s * (NCHUNK * ECH) + g * GE
        if gather:
            pltpu.make_async_copy(s4.at[pl.ds(off, GE)], isrc.at[p], sem_i).wait()
            for j in range(GE // 16):
                sl = pl.ds(j * 16, 16)
                isrc[p, sl] = isrc[p, sl] * FC + fofs
        pltpu.make_async_copy(
            dref.at[pl.ds(s * NCHUNK + g * G, G)], idst.at[p], sem_i
        ).wait()

    def scatter_edges(gather, fofs):
        def fire_gathers(p):
            return [
                pltpu.async_copy(
                    tab.at[isrc.at[p, pl.ds(b * ECH, ECH)]], rows.at[p, b], sem_g
                )
                for b in range(G)
            ]

        def fire_scatters(p):
            if not gather:
                return [
                    pltpu.async_copy(
                        rows.at[0, 0], spmem.at[idst.at[p, b]], sem_s, add=True
                    )
                    for b in range(G)
                ]
            return [
                pltpu.async_copy(
                    rows.at[p, b], spmem.at[idst.at[p, b]], sem_s, add=True
                )
                for b in range(G)
            ]

        def drain_scatters(p):
            for b in range(G):
                sref = rows.at[0, 0] if not gather else rows.at[p, b]
                pltpu.make_async_copy(sref, spmem.at[idst.at[p, b]], sem_s).wait()

        _load_idx(gather, 0, 0)

        def pair(k, _):
            g0 = 2 * k

            @pl.when(k > 0)
            def _():
                drain_scatters(1)

            _wait_idx(gather, fofs, g0, 0)
            gd0 = fire_gathers(0) if gather else []
            _load_idx(gather, g0 + 1, 1)
            for d in gd0:
                d.wait()
            sd0 = fire_scatters(0)
            _wait_idx(gather, fofs, g0 + 1, 1)
            gd1 = fire_gathers(1) if gather else []
            for d in sd0:
                d.wait()

            @pl.when(k < NPAIR - 1)
            def _():
                _load_idx(gather, g0 + 2, 0)

            for d in gd1:
                d.wait()
            fire_scatters(1)
            return 0

        lax.fori_loop(0, NPAIR, pair, 0)
        drain_scatters(1)

    def writeback(oref, colbase):
        pltpu.async_copy(
            spmem.at[pl.ds(base, RPS)],
            oref.at[pl.ds(base, RPS), pl.ds(colbase, FW)],
            sem_z,
        ).wait()

    phases = [(True, f, aggref, f * FW) for f in range(FC)]
    if with_cnt:
        phases.append((False, 0, cntref, 0))

    for pi, ph in enumerate(phases):

        @pl.when(c == (pi % NC))
        def _(ph=ph):
            gather, fofs, oref, colbase = ph
            zero_stripe()
            if not gather:
                lax.fori_loop(0, ECH, fill1, 0)
            plsc.subcore_barrier()
            scatter_edges(gather, fofs)
            plsc.subcore_barrier()
            writeback(oref, colbase)


def _make_sc_agg(with_cnt):
    out_type = [jax.ShapeDtypeStruct((NP, D), _f32)]
    if with_cnt:
        out_type += [jax.ShapeDtypeStruct((NP, FW), _f32)]
    mesh = plsc.VectorSubcoreMesh(
        core_axis_name="c", subcore_axis_name="s", num_cores=NC, num_subcores=NS
    )
    return pl.kernel(
        functools.partial(_sc_body, with_cnt),
        out_type=tuple(out_type),
        mesh=mesh,
        compiler_params=pltpu.CompilerParams(use_tc_tiling_on_sc=False),
        scratch_types=[
            pltpu.VMEM_SHARED((NP, FW), _f32),
            pltpu.VMEM((2, GE), _i32),
            pltpu.VMEM((2, G, ECH), _i32),
            pltpu.VMEM((2, G, ECH, FW), _f32),
            pltpu.SemaphoreType.DMA,
            pltpu.SemaphoreType.DMA,
            pltpu.SemaphoreType.DMA,
            pltpu.SemaphoreType.DMA,
        ],
    )


def _l1_body(a, cnt, x, wl, bl, wr, o):
    sacc = jnp.dot(a[...], wl[...], preferred_element_type=_f32)
    r = 1.0 / jnp.maximum(cnt[:, 0:1], 1.0)
    o[...] = jnp.maximum(
        sacc * r + bl[...] + jnp.dot(x[...], wr[...], preferred_element_type=_f32),
        0.0,
    )


def _tc_l1(agg, cntbuf, x, wl, bl, wr):
    return pl.pallas_call(
        _l1_body,
        grid=(GRID,),
        in_specs=[
            pl.BlockSpec((BR, D), lambda i: (i, 0)),
            pl.BlockSpec((BR, FW), lambda i: (i, 0)),
            pl.BlockSpec((BR, D), lambda i: (i, 0)),
            pl.BlockSpec((D, D), lambda i: (0, 0)),
            pl.BlockSpec((1, D), lambda i: (0, 0)),
            pl.BlockSpec((D, D), lambda i: (0, 0)),
        ],
        out_specs=pl.BlockSpec((BR, D), lambda i: (i, 0)),
        out_shape=jax.ShapeDtypeStruct((NP, D), _f32),
    )(agg, cntbuf, x, wl, bl, wr)


def _l2_body(a, cnt, x, wl, bl, wr, b3, o, acc, cacc):
    i = pl.program_id(0)

    @pl.when(i == 0)
    def _():
        acc[...] = jnp.zeros_like(acc)
        cacc[...] = jnp.zeros_like(cacc)

    sacc = jnp.dot(a[...], wl[...], preferred_element_type=_f32)
    r = 1.0 / jnp.maximum(cnt[:, 0:1], 1.0)
    h2 = sacc * r + bl[...] + jnp.dot(x[...], wr[...], preferred_element_type=_f32)
    gidx = i * BR + lax.broadcasted_iota(_i32, (BR, 1), 0)
    h2 = jnp.where(gidx < N, h2, 0.0)
    bids = b3[0]
    oh = (lax.broadcasted_iota(_i32, (NB, BR), 0) == bids).astype(_f32)
    acc[...] += jnp.dot(oh, h2, preferred_element_type=_f32)
    cacc[...] += jnp.sum(oh, axis=1, keepdims=True)

    @pl.when(i == GRID - 1)
    def _():
        o[...] = acc[...] / jnp.maximum(cacc[...], 1.0)


def _tc_l2pool(agg, cntbuf, x, wl, bl, wr, b3):
    return pl.pallas_call(
        _l2_body,
        grid=(GRID,),
        in_specs=[
            pl.BlockSpec((BR, D), lambda i: (i, 0)),
            pl.BlockSpec((BR, FW), lambda i: (i, 0)),
            pl.BlockSpec((BR, D), lambda i: (i, 0)),
            pl.BlockSpec((D, D), lambda i: (0, 0)),
            pl.BlockSpec((1, D), lambda i: (0, 0)),
            pl.BlockSpec((D, D), lambda i: (0, 0)),
            pl.BlockSpec((1, 1, BR), lambda i: (i, 0, 0)),
        ],
        out_specs=pl.BlockSpec((NB, D), lambda i: (0, 0)),
        out_shape=jax.ShapeDtypeStruct((NB, D), _f32),
        scratch_shapes=[
            pltpu.VMEM((NB, D), _f32),
            pltpu.VMEM((NB, D), _f32),
        ],
    )(agg, cntbuf, x, wl, bl, wr, b3)


def _head_body(pu, pi_, w1a, w1b, b1, w2, b2, o):
    x = pu[...] @ w1a[...] + pi_[...] @ w1b[...] + b1[...]
    x = jnp.maximum(x, 0.0)
    lg = x @ w2[...] + b2[...]
    m = jnp.max(lg, axis=1, keepdims=True)
    e = jnp.exp(lg - m)
    sm = jnp.sum(e, axis=1, keepdims=True)
    o[...] = lg - m - jnp.log(sm)


def _tc_head(pu, pi_, w1a, w1b, b1, w2p, b2p):
    return pl.pallas_call(
        _head_body,
        out_shape=jax.ShapeDtypeStruct((NB, D), _f32),
    )(pu, pi_, w1a, w1b, b1, w2p, b2p)


def _prep_edges(edge_index):
    src = edge_index[0].astype(_i32)
    dst = edge_index[1].astype(_i32)
    src = jnp.pad(src, (0, EPAD - E))
    dst = jnp.pad(dst, (0, EPAD - E), constant_values=NP - 1)
    return src, dst.reshape(EPAD // ECH, ECH)


def kernel(x_user, x_item, edge_index_u2i, edge_index_i2u, batch_user, batch_item,
           W1_ui_l, b1_ui_l, W1_ui_r, W1_iu_l, b1_iu_l, W1_iu_r,
           W2_ui_l, b2_ui_l, W2_ui_r, W2_iu_l, b2_iu_l, W2_iu_r,
           W_lin1, b_lin1, W_lin2, b_lin2):
    s4u, du = _prep_edges(edge_index_u2i)
    s4i, di = _prep_edges(edge_index_i2u)

    zs = jnp.zeros((RPS, FW), _f32)
    sc1 = _make_sc_agg(True)
    agg_ui, cnt_ui = sc1(x_user.reshape(FC * N, FW), s4u, du, zs)
    agg_iu, cnt_iu = sc1(x_item.reshape(FC * N, FW), s4i, di, zs)

    b1ui = b1_ui_l.reshape(1, D)
    b1iu = b1_iu_l.reshape(1, D)
    h_item = _tc_l1(agg_ui, cnt_ui, x_item, W1_ui_l, b1ui, W1_ui_r)
    h_user = _tc_l1(agg_iu, cnt_iu, x_user, W1_iu_l, b1iu, W1_iu_r)

    sc2 = _make_sc_agg(False)
    (agg2_ui,) = sc2(h_user.reshape(FC * NP, FW), s4u, du, zs)
    (agg2_iu,) = sc2(h_item.reshape(FC * NP, FW), s4i, di, zs)

    def b3(batch):
        b = jnp.pad(batch.astype(_i32), (0, NP - N), constant_values=NB)
        return b.reshape(GRID, 1, BR)

    p_item = _tc_l2pool(agg2_ui, cnt_ui, h_item, W2_ui_l,
                        b2_ui_l.reshape(1, D), W2_ui_r, b3(batch_item))
    p_user = _tc_l2pool(agg2_iu, cnt_iu, h_user, W2_iu_l,
                        b2_iu_l.reshape(1, D), W2_iu_r, b3(batch_user))

    w2p = jnp.pad(W_lin2, ((0, 0), (0, D - NCLS)))
    b2p = jnp.pad(b_lin2, (0, D - NCLS), constant_values=-1e30).reshape(1, D)
    out = _tc_head(p_user, p_item, W_lin1[:D], W_lin1[D:],
                   b_lin1.reshape(1, D), w2p, b2p)
    return out[:, :NCLS]


# speedup vs baseline: 1.0112x; 1.0112x over previous
import functools

import jax
import jax.numpy as jnp
from jax import lax
from jax.experimental import pallas as pl
from jax.experimental.pallas import tpu as pltpu
from jax.experimental.pallas import tpu_sc as plsc

N = 50000
D = 128
E = 500000
NB = 64
NCLS = 10
NC, NS = 2, 16
BR = 1024
NP = 50176
GRID = NP // BR
ECH = 128
G = 3
GE = G * ECH
NPAIR = 41
EPAD = NS * GE * 2 * NPAIR
NCHUNK = EPAD // (NS * ECH)
RPS = NP // NS
FC = 4
FW = 32

_f32 = jnp.float32
_i32 = jnp.int32


def _sc_body(with_cnt, *refs):
    nouts = 2 if with_cnt else 1
    (tab, s4, dref, zs) = refs[:4]
    outs = refs[4:4 + nouts]
    (spmem, isrc, idst, rows, sem_i, sem_g, sem_s, sem_z) = refs[4 + nouts:]
    aggref = outs[0]
    cntref = outs[1] if with_cnt else None

    c = lax.axis_index("c")
    s = lax.axis_index("s")
    base = s * RPS

    ov = jnp.ones((16,), _f32)

    def fill1(r, _):
        rows[0, 0, r, pl.ds(0, 16)] = ov
        rows[0, 0, r, pl.ds(16, 16)] = ov
        return 0

    def zero_stripe():
        pltpu.async_copy(zs, spmem.at[pl.ds(base, RPS)], sem_z).wait()

    def _load_idx(gather, g, p):
        off = s * (NCHUNK * ECH) + g * GE
        if gather:
            pltpu.async_copy(s4.at[pl.ds(off, GE)], isrc.at[p], sem_i)
        pltpu.async_copy(dref.at[pl.ds(s * NCHUNK + g * G, G)], idst.at[p], sem_i)

    def _wait_idx(gather, fofs, g, p):
        off = s * (NCHUNK * ECH) + g * GE
        if gather:
            pltpu.make_async_copy(s4.at[pl.ds(off, GE)], isrc.at[p], sem_i).wait()
            for j in range(GE // 16):
                sl = pl.ds(j * 16, 16)
                isrc[p, sl] = isrc[p, sl] * FC + fofs
        pltpu.make_async_copy(
            dref.at[pl.ds(s * NCHUNK + g * G, G)], idst.at[p], sem_i
        ).wait()

    def scatter_edges(gather, fofs):
        def fire_gathers(p):
            return [
                pltpu.async_copy(
                    tab.at[isrc.at[p, pl.ds(b * ECH, ECH)]], rows.at[p, b], sem_g
                )
                for b in range(G)
            ]

        def fire_scatters(p):
            if not gather:
                return [
                    pltpu.async_copy(
                        rows.at[0, 0], spmem.at[idst.at[p, b]], sem_s, add=True
                    )
                    for b in range(G)
                ]
            return [
                pltpu.async_copy(
                    rows.at[p, b], spmem.at[idst.at[p, b]], sem_s, add=True
                )
                for b in range(G)
            ]

        def drain_scatters(p):
            for b in range(G):
                sref = rows.at[0, 0] if not gather else rows.at[p, b]
                pltpu.make_async_copy(sref, spmem.at[idst.at[p, b]], sem_s).wait()

        _load_idx(gather, 0, 0)

        def pair(k, _):
            g0 = 2 * k

            @pl.when(k > 0)
            def _():
                drain_scatters(1)

            _wait_idx(gather, fofs, g0, 0)
            gd0 = fire_gathers(0) if gather else []
            _load_idx(gather, g0 + 1, 1)
            for d in gd0:
                d.wait()
            sd0 = fire_scatters(0)
            _wait_idx(gather, fofs, g0 + 1, 1)
            gd1 = fire_gathers(1) if gather else []
            for d in sd0:
                d.wait()

            @pl.when(k < NPAIR - 1)
            def _():
                _load_idx(gather, g0 + 2, 0)

            for d in gd1:
                d.wait()
            fire_scatters(1)
            return 0

        lax.fori_loop(0, NPAIR, pair, 0)
        drain_scatters(1)

    def writeback(oref, colbase):
        pltpu.async_copy(
            spmem.at[pl.ds(base, RPS)],
            oref.at[pl.ds(base, RPS), pl.ds(colbase, FW)],
            sem_z,
        ).wait()

    phases = [(True, f, aggref, f * FW) for f in range(FC)]
    if with_cnt:
        phases.append((False, 0, cntref, 0))

    for pi, ph in enumerate(phases):

        @pl.when(c == (pi % NC))
        def _(ph=ph):
            gather, fofs, oref, colbase = ph
            zero_stripe()
            if not gather:
                lax.fori_loop(0, ECH, fill1, 0)
            plsc.subcore_barrier()
            scatter_edges(gather, fofs)
            plsc.subcore_barrier()
            writeback(oref, colbase)


def _make_sc_agg(with_cnt):
    out_type = [jax.ShapeDtypeStruct((NP, D), _f32)]
    if with_cnt:
        out_type += [jax.ShapeDtypeStruct((NP, FW), _f32)]
    mesh = plsc.VectorSubcoreMesh(
        core_axis_name="c", subcore_axis_name="s", num_cores=NC, num_subcores=NS
    )
    return pl.kernel(
        functools.partial(_sc_body, with_cnt),
        out_type=tuple(out_type),
        mesh=mesh,
        compiler_params=pltpu.CompilerParams(use_tc_tiling_on_sc=False),
        scratch_types=[
            pltpu.VMEM_SHARED((NP, FW), _f32),
            pltpu.VMEM((2, GE), _i32),
            pltpu.VMEM((2, G, ECH), _i32),
            pltpu.VMEM((2, G, ECH, FW), _f32),
            pltpu.SemaphoreType.DMA,
            pltpu.SemaphoreType.DMA,
            pltpu.SemaphoreType.DMA,
            pltpu.SemaphoreType.DMA,
        ],
    )


def _l1_body(a, cnt, x, wl, bl, wr, o):
    sacc = jnp.dot(a[...], wl[...], preferred_element_type=_f32)
    r = 1.0 / jnp.maximum(cnt[:, 0:1], 1.0)
    o[...] = jnp.maximum(
        sacc * r + bl[...] + jnp.dot(x[...], wr[...], preferred_element_type=_f32),
        0.0,
    )


def _tc_l1(agg, cntbuf, x, wl, bl, wr):
    return pl.pallas_call(
        _l1_body,
        grid=(GRID,),
        in_specs=[
            pl.BlockSpec((BR, D), lambda i: (i, 0)),
            pl.BlockSpec((BR, FW), lambda i: (i, 0)),
            pl.BlockSpec((BR, D), lambda i: (i, 0)),
            pl.BlockSpec((D, D), lambda i: (0, 0)),
            pl.BlockSpec((1, D), lambda i: (0, 0)),
            pl.BlockSpec((D, D), lambda i: (0, 0)),
        ],
        out_specs=pl.BlockSpec((BR, D), lambda i: (i, 0)),
        out_shape=jax.ShapeDtypeStruct((NP, D), _f32),
    )(agg, cntbuf, x, wl, bl, wr)


def _l2_body(a, cnt, x, wl, bl, wr, b3, o, acc, cacc):
    i = pl.program_id(0)

    @pl.when(i == 0)
    def _():
        acc[...] = jnp.zeros_like(acc)
        cacc[...] = jnp.zeros_like(cacc)

    sacc = jnp.dot(a[...], wl[...], preferred_element_type=_f32)
    r = 1.0 / jnp.maximum(cnt[:, 0:1], 1.0)
    h2 = sacc * r + bl[...] + jnp.dot(x[...], wr[...], preferred_element_type=_f32)
    gidx = i * BR + lax.broadcasted_iota(_i32, (BR, 1), 0)
    h2 = jnp.where(gidx < N, h2, 0.0)
    bids = b3[0]
    oh = (lax.broadcasted_iota(_i32, (NB, BR), 0) == bids).astype(_f32)
    acc[...] += jnp.dot(oh, h2, preferred_element_type=_f32)
    cacc[...] += jnp.sum(oh, axis=1, keepdims=True)

    @pl.when(i == GRID - 1)
    def _():
        o[...] = acc[...] / jnp.maximum(cacc[...], 1.0)


def _tc_l2pool(agg, cntbuf, x, wl, bl, wr, b3):
    return pl.pallas_call(
        _l2_body,
        grid=(GRID,),
        in_specs=[
            pl.BlockSpec((BR, D), lambda i: (i, 0)),
            pl.BlockSpec((BR, FW), lambda i: (i, 0)),
            pl.BlockSpec((BR, D), lambda i: (i, 0)),
            pl.BlockSpec((D, D), lambda i: (0, 0)),
            pl.BlockSpec((1, D), lambda i: (0, 0)),
            pl.BlockSpec((D, D), lambda i: (0, 0)),
            pl.BlockSpec((1, 1, BR), lambda i: (i, 0, 0)),
        ],
        out_specs=pl.BlockSpec((NB, D), lambda i: (0, 0)),
        out_shape=jax.ShapeDtypeStruct((NB, D), _f32),
        scratch_shapes=[
            pltpu.VMEM((NB, D), _f32),
            pltpu.VMEM((NB, D), _f32),
        ],
    )(agg, cntbuf, x, wl, bl, wr, b3)


def _head_body(pu, pi_, w1a, w1b, b1, w2, b2, o):
    x = pu[...] @ w1a[...] + pi_[...] @ w1b[...] + b1[...]
    x = jnp.maximum(x, 0.0)
    lg = x @ w2[...] + b2[...]
    m = jnp.max(lg, axis=1, keepdims=True)
    e = jnp.exp(lg - m)
    sm = jnp.sum(e, axis=1, keepdims=True)
    o[...] = lg - m - jnp.log(sm)


def _tc_head(pu, pi_, w1a, w1b, b1, w2p, b2p):
    return pl.pallas_call(
        _head_body,
        out_shape=jax.ShapeDtypeStruct((NB, D), _f32),
    )(pu, pi_, w1a, w1b, b1, w2p, b2p)


def _prep_edges(edge_index):
    src = edge_index[0].astype(_i32)
    dst = edge_index[1].astype(_i32)
    src = jnp.pad(src, (0, EPAD - E))
    dst = jnp.pad(dst, (0, EPAD - E), constant_values=NP - 1)
    return src, dst.reshape(EPAD // ECH, ECH)


def kernel(x_user, x_item, edge_index_u2i, edge_index_i2u, batch_user, batch_item,
           W1_ui_l, b1_ui_l, W1_ui_r, W1_iu_l, b1_iu_l, W1_iu_r,
           W2_ui_l, b2_ui_l, W2_ui_r, W2_iu_l, b2_iu_l, W2_iu_r,
           W_lin1, b_lin1, W_lin2, b_lin2):
    s4u, du = _prep_edges(edge_index_u2i)
    s4i, di = _prep_edges(edge_index_i2u)

    zs = jnp.zeros((RPS, FW), _f32)
    sc1 = _make_sc_agg(True)
    agg_ui, cnt_ui = sc1(x_user.reshape(FC * N, FW), s4u, du, zs)
    agg_iu, cnt_iu = sc1(x_item.reshape(FC * N, FW), s4i, di, zs)

    b1ui = b1_ui_l.reshape(1, D)
    b1iu = b1_iu_l.reshape(1, D)
    h_item = _tc_l1(agg_ui, cnt_ui, x_item, W1_ui_l, b1ui, W1_ui_r)
    h_user = _tc_l1(agg_iu, cnt_iu, x_user, W1_iu_l, b1iu, W1_iu_r)

    sc2 = _make_sc_agg(False)
    (agg2_ui,) = sc2(h_user.reshape(FC * NP, FW), s4u, du, zs)
    (agg2_iu,) = sc2(h_item.reshape(FC * NP, FW), s4i, di, zs)

    def b3(batch):
        b = jnp.pad(batch.astype(_i32), (0, NP - N), constant_values=NB)
        return b.reshape(GRID, 1, BR)

    p_item = _tc_l2pool(agg2_ui, cnt_ui, h_item, W2_ui_l,
                        b2_ui_l.reshape(1, D), W2_ui_r, b3(batch_item))
    p_user = _tc_l2pool(agg2_iu, cnt_iu, h_user, W2_iu_l,
                        b2_iu_l.reshape(1, D), W2_iu_r, b3(batch_user))

    w2p = jnp.pad(W_lin2, ((0, 0), (0, D - NCLS)))
    b2p = jnp.pad(b_lin2, (0, D - NCLS), constant_values=-1e30).reshape(1, D)
    out = _tc_head(p_user, p_item, W_lin1[:D], W_lin1[D:],
                   b_lin1.reshape(1, D), w2p, b2p)
    return out[:, :NCLS]
